# baseline (speedup 1.0000x reference)
import functools

import jax
import jax.numpy as jnp
from jax import lax
from jax.experimental import pallas as pl
from jax.experimental.pallas import tpu as pltpu
from jax.experimental.pallas import tpu_sc as plsc

NC, NS, L = 2, 16, 16
NW = NC * NS
B, C, NBOX = 128, 4, 32768
RPW = B // NW
B_SC = NW
TC_R = 8
NEG_INF = -3.0e38

_MESH = plsc.VectorSubcoreMesh(
    core_axis_name="c", subcore_axis_name="s", num_cores=NC, num_subcores=NS)


def _insert3(m1, m2, m3, v):
  hi1 = jnp.maximum(m1, v)
  lo1 = jnp.minimum(m1, v)
  hi2 = jnp.maximum(m2, lo1)
  lo2 = jnp.minimum(m2, lo1)
  hi3 = jnp.maximum(m3, lo2)
  return hi1, hi2, hi3


def _softplus(v):
  return jnp.maximum(v, 0.0) + jnp.log1p(jnp.exp(-jnp.abs(v)))


def _sc_stage(class_preds, box_idxs, reg_preds, targ_flat):
  @functools.partial(
      pl.kernel,
      out_type=[
          jax.ShapeDtypeStruct((NW, L), jnp.float32),
          jax.ShapeDtypeStruct((B * C,), jnp.float32),
      ],
      mesh=_MESH,
      compiler_params=pltpu.CompilerParams(needs_layout_passes=False),
      scratch_types=[
          pltpu.VMEM((NBOX,), jnp.float32),
          pltpu.VMEM((B,), jnp.int32),
          pltpu.VMEM((L,), jnp.float32),
          pltpu.VMEM((RPW, C, 128), jnp.float32),
          pltpu.VMEM((L,), jnp.float32),
          pltpu.SemaphoreType.DMA,
          pltpu.SemaphoreType.DMA,
      ],
  )
  def body(class_hbm, box_hbm, reg_hbm, targ_hbm, cand_out, sl1_out,
           row_v, box_v, rot_v, regbuf_v, targ_v, sem, sem2):
    wid = lax.axis_index("s") * NC + lax.axis_index("c")
    lane = lax.iota(jnp.int32, L)

    row_cp = pltpu.async_copy(class_hbm.at[wid], row_v, sem)
    targ_cp = pltpu.async_copy(targ_hbm.at[pl.ds(wid * L, L)], targ_v, sem2)
    pltpu.sync_copy(box_hbm, box_v)

    reg_row0 = wid * RPW
    boxes = plsc.load_gather(box_v, [reg_row0 + (lane >> 2)])
    reg_copies = []
    for rl in range(RPW):
      posr = boxes[4 * rl]
      baser = pl.multiple_of((posr >> 7) << 7, 128)
      reg_copies.append(pltpu.async_copy(
          reg_hbm.at[reg_row0 + rl, :, pl.ds(baser, 128)],
          regbuf_v.at[rl], sem2))

    def rotate(x, s):
      rot_v[...] = x
      return plsc.load_gather(rot_v, [(lane + s) & (L - 1)])

    UN = 16
    NACC = 4
    pos = plsc.load_gather(box_v, [lane * 0 + wid])[0]
    row_cp.wait()
    base = (pos >> 4) << 4
    off = pos & (L - 1)
    vec = row_v[pl.ds(base, L)]
    is_pos = lane == off
    xpos = jnp.max(jnp.where(is_pos, vec, NEG_INF))
    row_v[pl.ds(base, L)] = jnp.where(is_pos, NEG_INF, vec)

    def scan_body(jj, carry):
      accs = [carry[3 * a:3 * a + 3] for a in range(NACC)]
      base_j = jj * (UN * L)
      for u in range(UN):
        v = row_v[pl.ds(base_j + u * L, L)]
        a = u % NACC
        accs[a] = _insert3(*accs[a], v)
      return tuple(x for acc in accs for x in acc)

    init = (jnp.full((L,), NEG_INF, jnp.float32),) * (3 * NACC)
    fin = lax.fori_loop(0, NBOX // (UN * L), scan_body, init)
    m1, m2, m3 = fin[0], fin[1], fin[2]
    for a in range(1, NACC):
      m1, m2, m3 = _insert3(m1, m2, m3, fin[3 * a])
      m1, m2, m3 = _insert3(m1, m2, m3, fin[3 * a + 1])
      m1, m2, m3 = _insert3(m1, m2, m3, fin[3 * a + 2])

    for s in (8, 4, 2, 1):
      n1 = rotate(m1, s)
      n2 = rotate(m2, s)
      n3 = rotate(m3, s)
      m1, m2, m3 = _insert3(m1, m2, m3, n1)
      m1, m2, m3 = _insert3(m1, m2, m3, n2)
      m1, m2, m3 = _insert3(m1, m2, m3, n3)

    stage = jnp.where(lane == 0, xpos, 0.0)
    stage = jnp.where(lane == 1, m1[0], stage)
    stage = jnp.where(lane == 2, m2[0], stage)
    stage = jnp.where(lane == 3, m3[0], stage)
    rot_v[...] = stage
    pltpu.sync_copy(rot_v, cand_out.at[wid])

    targ_cp.wait()
    for cp in reg_copies:
      cp.wait()
    val = plsc.load_gather(regbuf_v, [lane >> 2, lane & 3, boxes & 127])
    d = val - targ_v[...]
    ad = jnp.abs(d)
    targ_v[...] = jnp.where(ad < 1.0, 0.5 * d * d, ad - 0.5)
    pltpu.sync_copy(targ_v, sl1_out.at[pl.ds(wid * L, L)])

  return body(class_preds, box_idxs, reg_preds, targ_flat)


def _tc_scan(class_preds, box_col):
  nblk = (B - B_SC) // TC_R

  def body(x_ref, pos_ref, out_ref):
    i = pl.program_id(0)
    x = x_ref[...]
    pos = pos_ref[...]
    col = lax.broadcasted_iota(jnp.int32, x.shape, 1)
    ispos = col == pos
    xpos = jnp.sum(jnp.where(ispos, x, 0.0), axis=1, keepdims=True)
    xm = jnp.where(ispos, NEG_INF, x)
    t1 = jnp.max(xm, axis=1, keepdims=True)
    c1 = jnp.sum(jnp.where(xm == t1, 1.0, 0.0), axis=1, keepdims=True)
    xm2 = jnp.where(xm == t1, NEG_INF, xm)
    x2 = jnp.max(xm2, axis=1, keepdims=True)
    c2 = jnp.sum(jnp.where(xm2 == x2, 1.0, 0.0), axis=1, keepdims=True)
    xm3 = jnp.where(xm2 == x2, NEG_INF, xm2)
    x3 = jnp.max(xm3, axis=1, keepdims=True)
    v2 = jnp.where(c1 >= 2.0, t1, x2)
    v3 = jnp.where(c1 >= 3.0, t1, jnp.where(c1 + c2 >= 3.0, x2, x3))
    part = jnp.sum(_softplus(-xpos) + _softplus(t1)
                   + _softplus(v2) + _softplus(v3))

    @pl.when(i == 0)
    def _():
      out_ref[...] = jnp.zeros((1, 1), jnp.float32)

    out_ref[...] += part.reshape(1, 1)

  return pl.pallas_call(
      body,
      grid=(nblk,),
      in_specs=[
          pl.BlockSpec((TC_R, NBOX), lambda i: (B_SC // TC_R + i, 0)),
          pl.BlockSpec((TC_R, 1), lambda i: (B_SC // TC_R + i, 0)),
      ],
      out_specs=pl.BlockSpec((1, 1), lambda i: (0, 0)),
      out_shape=jax.ShapeDtypeStruct((1, 1), jnp.float32),
  )(class_preds, box_col)


def _tc_finish(cand, sl1, tc_part):
  def body(cand_ref, sl1_ref, part_ref, out_ref):
    x = cand_ref[...]
    col = lax.broadcasted_iota(jnp.int32, x.shape, 1)
    t = jnp.where(col == 0, 1.0, 0.0)
    bce = jnp.maximum(x, 0.0) - x * t + jnp.log1p(jnp.exp(-jnp.abs(x)))
    bce = jnp.where(col < 4, bce, 0.0)
    total = jnp.sum(bce) + part_ref[0, 0] + jnp.sum(sl1_ref[...])
    out_ref[...] = (total / jnp.float32(B * 4)).reshape(1, 1)

  return pl.pallas_call(
      body,
      out_shape=jax.ShapeDtypeStruct((1, 1), jnp.float32),
  )(cand, sl1.reshape(4, B), tc_part)


def kernel(class_preds, reg_preds, box_idxs, reg_targs):
  cand, sl1 = _sc_stage(
      class_preds,
      box_idxs,
      reg_preds,
      reg_targs.reshape(-1),
  )
  tc_part = _tc_scan(class_preds, box_idxs[:, None])
  return _tc_finish(cand, sl1, tc_part).reshape(())


# speedup vs baseline: 1.1365x; 1.1365x over previous
import functools

import jax
import jax.numpy as jnp
from jax import lax
from jax.experimental import pallas as pl
from jax.experimental.pallas import tpu as pltpu
from jax.experimental.pallas import tpu_sc as plsc

NC, NS, L = 2, 16, 16
NW = NC * NS
B, C, NBOX = 128, 4, 32768
RPW = B // NW
NEG_INF = -3.0e38

_MESH = plsc.VectorSubcoreMesh(
    core_axis_name="c", subcore_axis_name="s", num_cores=NC, num_subcores=NS)


def _insert3(m1, m2, m3, v):
  hi1 = jnp.maximum(m1, v)
  lo1 = jnp.minimum(m1, v)
  hi2 = jnp.maximum(m2, lo1)
  lo2 = jnp.minimum(m2, lo1)
  hi3 = jnp.maximum(m3, lo2)
  return hi1, hi2, hi3


def _sc_stage(class_preds, box_idxs, reg_preds, targ_flat):
  @functools.partial(
      pl.kernel,
      out_type=[
          jax.ShapeDtypeStruct((B * 4,), jnp.float32),
          jax.ShapeDtypeStruct((B * C,), jnp.float32),
      ],
      mesh=_MESH,
      compiler_params=pltpu.CompilerParams(needs_layout_passes=False),
      scratch_types=[
          pltpu.VMEM((NBOX,), jnp.float32),
          pltpu.VMEM((NBOX,), jnp.float32),
          pltpu.VMEM((NBOX,), jnp.float32),
          pltpu.VMEM((B,), jnp.int32),
          pltpu.VMEM((L,), jnp.float32),
          pltpu.VMEM((RPW, C, 128), jnp.float32),
          pltpu.VMEM((L,), jnp.float32),
          pltpu.SemaphoreType.DMA,
          pltpu.SemaphoreType.DMA,
      ],
  )
  def body(class_hbm, box_hbm, reg_hbm, targ_hbm, cand_out, sl1_out,
           row0_v, row1_v, row2_v, box_v, rot_v, regbuf_v, targ_v, sem, sem2):
    wid = lax.axis_index("s") * NC + lax.axis_index("c")
    lane = lax.iota(jnp.int32, L)
    bufs = (row0_v, row1_v, row2_v)
    row0 = wid * RPW

    copies = [None] * RPW
    for rl in range(3):
      copies[rl] = pltpu.async_copy(class_hbm.at[row0 + rl], bufs[rl], sem)
    targ_cp = pltpu.async_copy(targ_hbm.at[pl.ds(wid * L, L)], targ_v, sem2)
    pltpu.sync_copy(box_hbm, box_v)
    row_of_lane = row0 + (lane >> 2)
    boxes = plsc.load_gather(box_v, [row_of_lane])

    reg_copies = []
    for rl in range(RPW):
      posr = boxes[4 * rl]
      baser = pl.multiple_of((posr >> 7) << 7, 128)
      reg_copies.append(pltpu.async_copy(
          reg_hbm.at[row0 + rl, :, pl.ds(baser, 128)], regbuf_v.at[rl], sem2))

    def rotate(x, s):
      rot_v[...] = x
      return plsc.load_gather(rot_v, [(lane + s) & (L - 1)])

    UN = 16
    NACC = 4
    stage = jnp.zeros((L,), jnp.float32)
    for rl in range(RPW):
      if rl == 1:
        copies[3] = pltpu.async_copy(class_hbm.at[row0 + 3], bufs[0], sem)
      row_v = bufs[rl % 3]
      copies[rl].wait()
      pos = boxes[4 * rl]
      base = (pos >> 4) << 4
      off = pos & (L - 1)
      vec = row_v[pl.ds(base, L)]
      is_pos = lane == off
      xpos = jnp.max(jnp.where(is_pos, vec, NEG_INF))
      row_v[pl.ds(base, L)] = jnp.where(is_pos, NEG_INF, vec)

      def scan_body(jj, carry):
        accs = [carry[3 * a:3 * a + 3] for a in range(NACC)]
        base_j = jj * (UN * L)
        for u in range(UN):
          v = row_v[pl.ds(base_j + u * L, L)]
          a = u % NACC
          accs[a] = _insert3(*accs[a], v)
        return tuple(x for acc in accs for x in acc)

      init = (jnp.full((L,), NEG_INF, jnp.float32),) * (3 * NACC)
      fin = lax.fori_loop(0, NBOX // (UN * L), scan_body, init)
      m1, m2, m3 = fin[0], fin[1], fin[2]
      for a in range(1, NACC):
        m1, m2, m3 = _insert3(m1, m2, m3, fin[3 * a])
        m1, m2, m3 = _insert3(m1, m2, m3, fin[3 * a + 1])
        m1, m2, m3 = _insert3(m1, m2, m3, fin[3 * a + 2])

      for s in (8, 4, 2, 1):
        n1 = rotate(m1, s)
        n2 = rotate(m2, s)
        n3 = rotate(m3, s)
        m1, m2, m3 = _insert3(m1, m2, m3, n1)
        m1, m2, m3 = _insert3(m1, m2, m3, n2)
        m1, m2, m3 = _insert3(m1, m2, m3, n3)
      t1, t2, t3 = m1[0], m2[0], m3[0]

      stage = jnp.where(lane == 4 * rl, xpos, stage)
      stage = jnp.where(lane == 4 * rl + 1, t1, stage)
      stage = jnp.where(lane == 4 * rl + 2, t2, stage)
      stage = jnp.where(lane == 4 * rl + 3, t3, stage)

    rot_v[...] = stage
    pltpu.sync_copy(rot_v, cand_out.at[pl.ds(wid * L, L)])

    targ_cp.wait()
    for cp in reg_copies:
      cp.wait()
    val = plsc.load_gather(regbuf_v, [lane >> 2, lane & 3, boxes & 127])
    d = val - targ_v[...]
    ad = jnp.abs(d)
    targ_v[...] = jnp.where(ad < 1.0, 0.5 * d * d, ad - 0.5)
    pltpu.sync_copy(targ_v, sl1_out.at[pl.ds(wid * L, L)])

  return body(class_preds, box_idxs, reg_preds, targ_flat)


def _tc_finish(cand, sl1):
  def body(cand_ref, sl1_ref, out_ref):
    x = cand_ref[...]
    col = lax.broadcasted_iota(jnp.int32, x.shape, 1)
    t = jnp.where(col % 4 == 0, 1.0, 0.0)
    bce = jnp.maximum(x, 0.0) - x * t + jnp.log1p(jnp.exp(-jnp.abs(x)))
    total = jnp.sum(bce) + jnp.sum(sl1_ref[...])
    out_ref[...] = (total / jnp.float32(B * 4)).reshape(1, 1)

  out = pl.pallas_call(
      body,
      out_shape=jax.ShapeDtypeStruct((1, 1), jnp.float32),
  )(cand.reshape(4, B), sl1.reshape(4, B))
  return out


def kernel(class_preds, reg_preds, box_idxs, reg_targs):
  cand, sl1 = _sc_stage(
      class_preds,
      box_idxs,
      reg_preds,
      reg_targs.reshape(-1),
  )
  return _tc_finish(cand, sl1).reshape(())
